# Initial kernel scaffold; baseline (speedup 1.0000x reference)
import jax
import jax.numpy as jnp
from jax.experimental import pallas as pl


def kernel(x, edge_attr, params, edge_index, batch):
    raise NotImplementedError("write your pallas kernel here")


# speedup vs baseline: 2.9028x; 2.9028x over previous
import functools

import jax
import jax.numpy as jnp
from jax import lax
from jax.experimental import pallas as pl
from jax.experimental.pallas import tpu as pltpu
from jax.experimental.pallas import tpu_sc as plsc

EMB = 128
NL = 5
G = 64
NCLS = 10
N = 10000
E = 320000

NC = 2
NS = 16
NW = NC * NS
EPW = E // NW
CHUNK = 80
NCHUNK = EPW // CHUNK
NP = 10240
ZROWS = 64
RPS = NP // NS

_BN_INV = 1.0 / (1.0 + 1e-5) ** 0.5

R = 1000
GRID = N // R


def _edge_phase(h, ea, src, dst):
    mesh = plsc.VectorSubcoreMesh(core_axis_name="c", subcore_axis_name="s")

    @functools.partial(
        pl.kernel,
        out_type=jax.ShapeDtypeStruct((NC, NP, EMB), jnp.float32),
        mesh=mesh,
        scratch_types=[
            pltpu.VMEM((CHUNK,), jnp.int32),
            pltpu.VMEM((CHUNK,), jnp.int32),
            pltpu.VMEM((CHUNK, EMB), jnp.float32),
            pltpu.VMEM((CHUNK, EMB), jnp.float32),
            pltpu.VMEM((ZROWS, EMB), jnp.float32),
            pltpu.VMEM_SHARED((NP, EMB), jnp.float32),
            pltpu.SemaphoreType.DMA,
        ],
    )
    def k(h_hbm, ea_hbm, src_hbm, dst_hbm, out_hbm,
          src_v, dst_v, hrow_v, ea_v, zb_v, agg_sh, sem):
        c = lax.axis_index("c")
        s = lax.axis_index("s")
        wid = s * NC + c

        def zrow(r, _):
            for j in range(EMB // 16):
                zb_v[r, pl.ds(j * 16, 16)] = jnp.zeros((16,), jnp.float32)
            return 0
        lax.fori_loop(0, ZROWS, zrow, 0)
        for t in range(RPS // ZROWS):
            pltpu.sync_copy(zb_v, agg_sh.at[pl.ds(s * RPS + t * ZROWS, ZROWS)])
        plsc.subcore_barrier()

        def chunk(i, _):
            base = wid * EPW + i * CHUNK
            pltpu.sync_copy(src_hbm.at[pl.ds(base, CHUNK)], src_v)
            pltpu.sync_copy(dst_hbm.at[pl.ds(base, CHUNK)], dst_v)
            pltpu.async_copy(h_hbm.at[src_v], hrow_v, sem).wait()
            pltpu.sync_copy(ea_hbm.at[pl.ds(base, CHUNK)], ea_v)

            def row(r, _):
                for j in range(EMB // 16):
                    sl = pl.ds(j * 16, 16)
                    hrow_v[r, sl] = jnp.maximum(hrow_v[r, sl] + ea_v[r, sl], 0.0)
                return 0
            lax.fori_loop(0, CHUNK, row, 0)
            pltpu.sync_copy(hrow_v, agg_sh.at[dst_v], add=True)
            return 0
        lax.fori_loop(0, NCHUNK, chunk, 0)
        plsc.subcore_barrier()

        pltpu.sync_copy(agg_sh.at[pl.ds(s * RPS, RPS)],
                        out_hbm.at[c, pl.ds(s * RPS, RPS)])

    return k(h, ea, src, dst)[:, :N]


def _tc_prep(h_prev, vn, batch3):
    def body(h_ref, vn_ref, b_ref, h_out, pooled_out):
        i = pl.program_id(0)
        iota = lax.broadcasted_iota(jnp.int32, (G, 1), 0)
        onehot_t = (b_ref[0] == iota).astype(jnp.float32)
        hv = h_ref[...]
        h_out[...] = hv + lax.dot_general(
            onehot_t, vn_ref[...], (((0,), (0,)), ((), ())),
            preferred_element_type=jnp.float32)
        p = jnp.dot(onehot_t, hv, preferred_element_type=jnp.float32)

        @pl.when(i == 0)
        def _():
            pooled_out[...] = p

        @pl.when(i > 0)
        def _():
            pooled_out[...] += p

    return pl.pallas_call(
        body,
        grid=(GRID,),
        in_specs=[
            pl.BlockSpec((R, EMB), lambda i: (i, 0)),
            pl.BlockSpec((G, EMB), lambda i: (0, 0)),
            pl.BlockSpec((1, 1, R), lambda i: (i, 0, 0)),
        ],
        out_specs=[
            pl.BlockSpec((R, EMB), lambda i: (i, 0)),
            pl.BlockSpec((G, EMB), lambda i: (0, 0)),
        ],
        out_shape=[
            jax.ShapeDtypeStruct((N, EMB), jnp.float32),
            jax.ShapeDtypeStruct((G, EMB), jnp.float32),
        ],
    )(h_prev, vn, batch3)


def _tc_mlp(h, agg, eps11, c, relu_last):
    def body(h_ref, a_ref, e_ref, w1_ref, b1_ref, g1_ref, bb1_ref,
             w2_ref, b2_ref, g2_ref, bb2_ref, out_ref):
        t = (1.0 + e_ref[0, 0]) * h_ref[...] + a_ref[0] + a_ref[1]
        y = jnp.dot(t, w1_ref[...], preferred_element_type=jnp.float32)
        y = y + b1_ref[...]
        y = y * (g1_ref[...] * _BN_INV) + bb1_ref[...]
        y = jnp.maximum(y, 0.0)
        z = jnp.dot(y, w2_ref[...], preferred_element_type=jnp.float32)
        z = z + b2_ref[...]
        z = z * (g2_ref[...] * _BN_INV) + bb2_ref[...]
        if relu_last:
            z = jnp.maximum(z, 0.0)
        out_ref[...] = z

    return pl.pallas_call(
        body,
        grid=(GRID,),
        in_specs=[
            pl.BlockSpec((R, EMB), lambda i: (i, 0)),
            pl.BlockSpec((NC, R, EMB), lambda i: (0, i, 0)),
            pl.BlockSpec((1, 1), lambda i: (0, 0)),
            pl.BlockSpec((EMB, 2 * EMB), lambda i: (0, 0)),
            pl.BlockSpec((1, 2 * EMB), lambda i: (0, 0)),
            pl.BlockSpec((1, 2 * EMB), lambda i: (0, 0)),
            pl.BlockSpec((1, 2 * EMB), lambda i: (0, 0)),
            pl.BlockSpec((2 * EMB, EMB), lambda i: (0, 0)),
            pl.BlockSpec((1, EMB), lambda i: (0, 0)),
            pl.BlockSpec((1, EMB), lambda i: (0, 0)),
            pl.BlockSpec((1, EMB), lambda i: (0, 0)),
        ],
        out_specs=pl.BlockSpec((R, EMB), lambda i: (i, 0)),
        out_shape=jax.ShapeDtypeStruct((N, EMB), jnp.float32),
    )(h, agg, eps11,
      c['W1'], c['b1'].reshape(1, -1), c['bn1_g'].reshape(1, -1),
      c['bn1_b'].reshape(1, -1),
      c['W2'], c['b2'].reshape(1, -1), c['bn_g'].reshape(1, -1),
      c['bn_b'].reshape(1, -1))


def _tc_vn(pooled, vn, m):
    def body(p_ref, v_ref, w1_ref, b1_ref, g1_ref, bb1_ref,
             w2_ref, b2_ref, g2_ref, bb2_ref, out_ref):
        t = p_ref[...] + v_ref[...]
        y = jnp.dot(t, w1_ref[...], preferred_element_type=jnp.float32)
        y = y + b1_ref[...]
        y = jnp.maximum(y * (g1_ref[...] * _BN_INV) + bb1_ref[...], 0.0)
        z = jnp.dot(y, w2_ref[...], preferred_element_type=jnp.float32)
        z = z + b2_ref[...]
        out_ref[...] = jnp.maximum(z * (g2_ref[...] * _BN_INV) + bb2_ref[...], 0.0)

    return pl.pallas_call(
        body,
        out_shape=jax.ShapeDtypeStruct((G, EMB), jnp.float32),
    )(pooled, vn,
      m['W1'], m['b1'].reshape(1, -1), m['bn1_g'].reshape(1, -1),
      m['bn1_b'].reshape(1, -1),
      m['W2'], m['b2'].reshape(1, -1), m['bn2_g'].reshape(1, -1),
      m['bn2_b'].reshape(1, -1))


def _tc_final(h5, batch3, pooled_list, wf_pad, bf_pad):
    def body(h_ref, b_ref, p0, p1, p2, p3, p4, wf_ref, bf_ref,
             ne0, ne1, ne2, ne3, ne4, ne5, hg_ref, pred_ref,
             acc, cnt):
        i = pl.program_id(0)
        iota = lax.broadcasted_iota(jnp.int32, (G, 1), 0)
        onehot_t = (b_ref[0] == iota).astype(jnp.float32)
        p = jnp.dot(onehot_t, h_ref[...], preferred_element_type=jnp.float32)
        csum = jnp.broadcast_to(jnp.sum(onehot_t, axis=1, keepdims=True),
                                (G, EMB))

        @pl.when(i == 0)
        def _():
            acc[...] = p
            cnt[...] = csum

        @pl.when(i > 0)
        def _():
            acc[...] += p
            cnt[...] += csum

        @pl.when(i == GRID - 1)
        def _():
            inv = 1.0 / jnp.maximum(cnt[...], 1.0)
            ne0[...] = p0[...] * inv
            ne1[...] = p1[...] * inv
            ne2[...] = p2[...] * inv
            ne3[...] = p3[...] * inv
            ne4[...] = p4[...] * inv
            hg = acc[...] * inv
            ne5[...] = hg
            hg_ref[...] = hg
            pred_ref[...] = jnp.dot(hg, wf_ref[...],
                                    preferred_element_type=jnp.float32) + bf_ref[...]

    full = pl.BlockSpec((G, EMB), lambda i: (0, 0))
    return pl.pallas_call(
        body,
        grid=(GRID,),
        in_specs=[
            pl.BlockSpec((R, EMB), lambda i: (i, 0)),
            pl.BlockSpec((1, 1, R), lambda i: (i, 0, 0)),
            full, full, full, full, full,
            pl.BlockSpec((EMB, EMB), lambda i: (0, 0)),
            pl.BlockSpec((1, EMB), lambda i: (0, 0)),
        ],
        out_specs=[full] * 6 + [full, pl.BlockSpec((G, EMB), lambda i: (0, 0))],
        out_shape=[jax.ShapeDtypeStruct((G, EMB), jnp.float32)] * 7
        + [jax.ShapeDtypeStruct((G, EMB), jnp.float32)],
        scratch_shapes=[
            pltpu.VMEM((G, EMB), jnp.float32),
            pltpu.VMEM((G, EMB), jnp.float32),
        ],
    )(h5, batch3, *pooled_list, wf_pad, bf_pad)


def kernel(x, edge_attr, params, edge_index, batch):
    src = edge_index[0]
    dst = edge_index[1]
    batch3 = batch.reshape(GRID, 1, R)
    vn = jnp.broadcast_to(params['vn_emb'], (G, EMB))
    wf_pad = jnp.zeros((EMB, EMB), jnp.float32).at[:, :NCLS].set(params['Wf'])
    bf_pad = jnp.zeros((1, EMB), jnp.float32).at[0, :NCLS].set(params['bf'])

    h = x
    pooled = []
    for l in range(NL):
        c = params['convs'][l]
        h_in, p = _tc_prep(h, vn, batch3)
        pooled.append(p)
        agg = _edge_phase(h_in, edge_attr, src, dst)
        if l < NL - 1:
            vn = _tc_vn(p, vn, params['vnmlps'][l])
        h = _tc_mlp(h_in, agg, c['eps'].reshape(1, 1), c, relu_last=(l < NL - 1))

    outs = _tc_final(h, batch3, pooled, wf_pad, bf_pad)
    ne = tuple(outs[:6])
    h_graph = outs[6]
    pred = outs[7][:, :NCLS]
    return (pred, h_graph, ne)


# speedup vs baseline: 6.2821x; 2.1641x over previous
import functools

import jax
import jax.numpy as jnp
from jax import lax
from jax.experimental import pallas as pl
from jax.experimental.pallas import tpu as pltpu
from jax.experimental.pallas import tpu_sc as plsc

EMB = 128
NL = 5
G = 64
NCLS = 10
N = 10000
E = 320000

NC = 2
NS = 16
NW = NC * NS
EPW = E // NW
CHUNK = 40
NCHUNK = EPW // CHUNK
NPAIR = NCHUNK // 2
NP = 10240
ZROWS = 16
RPS = NP // NS

_BN_INV = 1.0 / (1.0 + 1e-5) ** 0.5

R = 1000
GRID = N // R


def _edge_phase(h, ea, src2, dst2):
    mesh = plsc.VectorSubcoreMesh(core_axis_name="c", subcore_axis_name="s")

    @functools.partial(
        pl.kernel,
        out_type=jax.ShapeDtypeStruct((NC, NP, EMB), jnp.float32),
        mesh=mesh,
        scratch_types=[
            pltpu.VMEM((2, CHUNK), jnp.int32),
            pltpu.VMEM((2, CHUNK), jnp.int32),
            pltpu.VMEM((2, CHUNK, EMB), jnp.float32),
            pltpu.VMEM((2, CHUNK, EMB), jnp.float32),
            pltpu.VMEM((ZROWS, EMB), jnp.float32),
            pltpu.VMEM_SHARED((NP, EMB), jnp.float32),
            pltpu.SemaphoreType.DMA((2,)),
            pltpu.SemaphoreType.DMA((2,)),
            pltpu.SemaphoreType.DMA((2,)),
            pltpu.SemaphoreType.DMA((2,)),
            pltpu.SemaphoreType.DMA,
        ],
    )
    def k(h_hbm, ea_hbm, src_hbm, dst_hbm, out_hbm,
          src_v, dst_v, hrow_v, ea_v, zb_v, agg_sh,
          ssem, dsem, gsem, esem, zsem):
        c = lax.axis_index("c")
        s = lax.axis_index("s")
        wid = s * NC + c
        ebase = wid * EPW

        def fire_src(i, b):
            pltpu.async_copy(src_hbm.at[wid, i], src_v.at[b], ssem.at[b])

        def fire_dst(i, b):
            pltpu.async_copy(dst_hbm.at[wid, i], dst_v.at[b], dsem.at[b])

        def wait_idx(sem, b):
            pltpu.make_async_copy(src_hbm.at[0, 0], src_v.at[b],
                                  sem.at[b]).wait()

        def fire_data(i, b):
            pltpu.async_copy(h_hbm.at[src_v.at[b]], hrow_v.at[b], gsem.at[b])
            pltpu.async_copy(ea_hbm.at[pl.ds(ebase + i * CHUNK, CHUNK)],
                             ea_v.at[b], esem.at[b])

        def wait_data(b):
            pltpu.make_async_copy(h_hbm.at[src_v.at[b]], hrow_v.at[b],
                                  gsem.at[b]).wait()
            pltpu.make_async_copy(ea_hbm.at[pl.ds(ebase, CHUNK)],
                                  ea_v.at[b], esem.at[b]).wait()

        fire_src(0, 0)
        fire_src(1, 1)
        fire_dst(0, 0)
        fire_dst(1, 1)

        def zrow(r, _):
            for j in range(EMB // 16):
                zb_v[r, pl.ds(j * 16, 16)] = jnp.zeros((16,), jnp.float32)
            return 0
        lax.fori_loop(0, ZROWS, zrow, 0)
        zd = []
        for t in range(RPS // ZROWS):
            zd.append(pltpu.async_copy(
                zb_v, agg_sh.at[pl.ds(s * RPS + t * ZROWS, ZROWS)], zsem))
        for d in zd:
            d.wait()
        plsc.subcore_barrier()

        wait_idx(ssem, 0)
        fire_data(0, 0)
        wait_idx(ssem, 1)
        fire_data(1, 1)

        def pair(g, _):
            for b in range(2):
                i = 2 * g + b
                inext = jnp.minimum(i + 2, NCHUNK - 1)
                wait_data(b)
                fire_src(inext, b)

                def row(r, _):
                    for j in range(EMB // 16):
                        sl = pl.ds(j * 16, 16)
                        hrow_v[b, r, sl] = jnp.maximum(
                            hrow_v[b, r, sl] + ea_v[b, r, sl], 0.0)
                    return 0
                lax.fori_loop(0, CHUNK, row, 0)
                wait_idx(dsem, b)
                pltpu.sync_copy(hrow_v.at[b], agg_sh.at[dst_v.at[b]], add=True)
                fire_dst(inext, b)
                wait_idx(ssem, b)
                fire_data(inext, b)
            return 0
        lax.fori_loop(0, NPAIR, pair, 0)
        for b in range(2):
            wait_data(b)
            wait_idx(dsem, b)
        plsc.subcore_barrier()

        pltpu.sync_copy(agg_sh.at[pl.ds(s * RPS, RPS)],
                        out_hbm.at[c, pl.ds(s * RPS, RPS)])

    return k(h, ea, src2, dst2)[:, :N]


def _tc_prep(h_prev, vn, batch3):
    def body(h_ref, vn_ref, b_ref, h_out, pooled_out):
        i = pl.program_id(0)
        iota = lax.broadcasted_iota(jnp.int32, (G, 1), 0)
        onehot_t = (b_ref[0] == iota).astype(jnp.float32)
        hv = h_ref[...]
        h_out[...] = hv + lax.dot_general(
            onehot_t, vn_ref[...], (((0,), (0,)), ((), ())),
            preferred_element_type=jnp.float32)
        p = jnp.dot(onehot_t, hv, preferred_element_type=jnp.float32)

        @pl.when(i == 0)
        def _():
            pooled_out[...] = p

        @pl.when(i > 0)
        def _():
            pooled_out[...] += p

    return pl.pallas_call(
        body,
        grid=(GRID,),
        in_specs=[
            pl.BlockSpec((R, EMB), lambda i: (i, 0)),
            pl.BlockSpec((G, EMB), lambda i: (0, 0)),
            pl.BlockSpec((1, 1, R), lambda i: (i, 0, 0)),
        ],
        out_specs=[
            pl.BlockSpec((R, EMB), lambda i: (i, 0)),
            pl.BlockSpec((G, EMB), lambda i: (0, 0)),
        ],
        out_shape=[
            jax.ShapeDtypeStruct((N, EMB), jnp.float32),
            jax.ShapeDtypeStruct((G, EMB), jnp.float32),
        ],
    )(h_prev, vn, batch3)


def _tc_mlp(h, agg, eps11, c, relu_last):
    def body(h_ref, a_ref, e_ref, w1_ref, b1_ref, g1_ref, bb1_ref,
             w2_ref, b2_ref, g2_ref, bb2_ref, out_ref):
        t = (1.0 + e_ref[0, 0]) * h_ref[...] + a_ref[0] + a_ref[1]
        y = jnp.dot(t, w1_ref[...], preferred_element_type=jnp.float32)
        y = y + b1_ref[...]
        y = y * (g1_ref[...] * _BN_INV) + bb1_ref[...]
        y = jnp.maximum(y, 0.0)
        z = jnp.dot(y, w2_ref[...], preferred_element_type=jnp.float32)
        z = z + b2_ref[...]
        z = z * (g2_ref[...] * _BN_INV) + bb2_ref[...]
        if relu_last:
            z = jnp.maximum(z, 0.0)
        out_ref[...] = z

    return pl.pallas_call(
        body,
        grid=(GRID,),
        in_specs=[
            pl.BlockSpec((R, EMB), lambda i: (i, 0)),
            pl.BlockSpec((NC, R, EMB), lambda i: (0, i, 0)),
            pl.BlockSpec((1, 1), lambda i: (0, 0)),
            pl.BlockSpec((EMB, 2 * EMB), lambda i: (0, 0)),
            pl.BlockSpec((1, 2 * EMB), lambda i: (0, 0)),
            pl.BlockSpec((1, 2 * EMB), lambda i: (0, 0)),
            pl.BlockSpec((1, 2 * EMB), lambda i: (0, 0)),
            pl.BlockSpec((2 * EMB, EMB), lambda i: (0, 0)),
            pl.BlockSpec((1, EMB), lambda i: (0, 0)),
            pl.BlockSpec((1, EMB), lambda i: (0, 0)),
            pl.BlockSpec((1, EMB), lambda i: (0, 0)),
        ],
        out_specs=pl.BlockSpec((R, EMB), lambda i: (i, 0)),
        out_shape=jax.ShapeDtypeStruct((N, EMB), jnp.float32),
    )(h, agg, eps11,
      c['W1'], c['b1'].reshape(1, -1), c['bn1_g'].reshape(1, -1),
      c['bn1_b'].reshape(1, -1),
      c['W2'], c['b2'].reshape(1, -1), c['bn_g'].reshape(1, -1),
      c['bn_b'].reshape(1, -1))


def _tc_vn(pooled, vn, m):
    def body(p_ref, v_ref, w1_ref, b1_ref, g1_ref, bb1_ref,
             w2_ref, b2_ref, g2_ref, bb2_ref, out_ref):
        t = p_ref[...] + v_ref[...]
        y = jnp.dot(t, w1_ref[...], preferred_element_type=jnp.float32)
        y = y + b1_ref[...]
        y = jnp.maximum(y * (g1_ref[...] * _BN_INV) + bb1_ref[...], 0.0)
        z = jnp.dot(y, w2_ref[...], preferred_element_type=jnp.float32)
        z = z + b2_ref[...]
        out_ref[...] = jnp.maximum(z * (g2_ref[...] * _BN_INV) + bb2_ref[...], 0.0)

    return pl.pallas_call(
        body,
        out_shape=jax.ShapeDtypeStruct((G, EMB), jnp.float32),
    )(pooled, vn,
      m['W1'], m['b1'].reshape(1, -1), m['bn1_g'].reshape(1, -1),
      m['bn1_b'].reshape(1, -1),
      m['W2'], m['b2'].reshape(1, -1), m['bn2_g'].reshape(1, -1),
      m['bn2_b'].reshape(1, -1))


def _tc_final(h5, batch3, pooled_list, wf_pad, bf_pad):
    def body(h_ref, b_ref, p0, p1, p2, p3, p4, wf_ref, bf_ref,
             ne0, ne1, ne2, ne3, ne4, ne5, hg_ref, pred_ref,
             acc, cnt):
        i = pl.program_id(0)
        iota = lax.broadcasted_iota(jnp.int32, (G, 1), 0)
        onehot_t = (b_ref[0] == iota).astype(jnp.float32)
        p = jnp.dot(onehot_t, h_ref[...], preferred_element_type=jnp.float32)
        csum = jnp.broadcast_to(jnp.sum(onehot_t, axis=1, keepdims=True),
                                (G, EMB))

        @pl.when(i == 0)
        def _():
            acc[...] = p
            cnt[...] = csum

        @pl.when(i > 0)
        def _():
            acc[...] += p
            cnt[...] += csum

        @pl.when(i == GRID - 1)
        def _():
            inv = 1.0 / jnp.maximum(cnt[...], 1.0)
            ne0[...] = p0[...] * inv
            ne1[...] = p1[...] * inv
            ne2[...] = p2[...] * inv
            ne3[...] = p3[...] * inv
            ne4[...] = p4[...] * inv
            hg = acc[...] * inv
            ne5[...] = hg
            hg_ref[...] = hg
            pred_ref[...] = jnp.dot(hg, wf_ref[...],
                                    preferred_element_type=jnp.float32) + bf_ref[...]

    full = pl.BlockSpec((G, EMB), lambda i: (0, 0))
    return pl.pallas_call(
        body,
        grid=(GRID,),
        in_specs=[
            pl.BlockSpec((R, EMB), lambda i: (i, 0)),
            pl.BlockSpec((1, 1, R), lambda i: (i, 0, 0)),
            full, full, full, full, full,
            pl.BlockSpec((EMB, EMB), lambda i: (0, 0)),
            pl.BlockSpec((1, EMB), lambda i: (0, 0)),
        ],
        out_specs=[full] * 6 + [full, pl.BlockSpec((G, EMB), lambda i: (0, 0))],
        out_shape=[jax.ShapeDtypeStruct((G, EMB), jnp.float32)] * 7
        + [jax.ShapeDtypeStruct((G, EMB), jnp.float32)],
        scratch_shapes=[
            pltpu.VMEM((G, EMB), jnp.float32),
            pltpu.VMEM((G, EMB), jnp.float32),
        ],
    )(h5, batch3, *pooled_list, wf_pad, bf_pad)


def kernel(x, edge_attr, params, edge_index, batch):
    src = edge_index[0].reshape(NW, NCHUNK, CHUNK)
    dst = edge_index[1].reshape(NW, NCHUNK, CHUNK)
    batch3 = batch.reshape(GRID, 1, R)
    vn = jnp.broadcast_to(params['vn_emb'], (G, EMB))
    wf_pad = jnp.zeros((EMB, EMB), jnp.float32).at[:, :NCLS].set(params['Wf'])
    bf_pad = jnp.zeros((1, EMB), jnp.float32).at[0, :NCLS].set(params['bf'])

    h = x
    pooled = []
    for l in range(NL):
        c = params['convs'][l]
        h_in, p = _tc_prep(h, vn, batch3)
        pooled.append(p)
        agg = _edge_phase(h_in, edge_attr, src, dst)
        if l < NL - 1:
            vn = _tc_vn(p, vn, params['vnmlps'][l])
        h = _tc_mlp(h_in, agg, c['eps'].reshape(1, 1), c, relu_last=(l < NL - 1))

    outs = _tc_final(h, batch3, pooled, wf_pad, bf_pad)
    ne = tuple(outs[:6])
    h_graph = outs[6]
    pred = outs[7][:, :NCLS]
    return (pred, h_graph, ne)


# speedup vs baseline: 6.4631x; 1.0288x over previous
import functools

import jax
import jax.numpy as jnp
from jax import lax
from jax.experimental import pallas as pl
from jax.experimental.pallas import tpu as pltpu
from jax.experimental.pallas import tpu_sc as plsc

EMB = 128
NL = 5
G = 64
NCLS = 10
N = 10000
E = 320000

NC = 2
NS = 16
NW = NC * NS
EPW = E // NW
CHUNK = 40
NCHUNK = EPW // CHUNK
NPAIR = NCHUNK // 2
NP = 10240
ZROWS = 16
RPS = NP // NS

_BN_INV = 1.0 / (1.0 + 1e-5) ** 0.5

R = 1000
GRID = N // R


def _edge_phase(h, ea, src2, dst2):
    mesh = plsc.VectorSubcoreMesh(core_axis_name="c", subcore_axis_name="s")

    @functools.partial(
        pl.kernel,
        out_type=jax.ShapeDtypeStruct((NC, NP, EMB), jnp.float32),
        mesh=mesh,
        scratch_types=[
            pltpu.VMEM((2, CHUNK), jnp.int32),
            pltpu.VMEM((2, CHUNK), jnp.int32),
            pltpu.VMEM((2, CHUNK, EMB), jnp.float32),
            pltpu.VMEM((2, CHUNK, EMB), jnp.float32),
            pltpu.VMEM((ZROWS, EMB), jnp.float32),
            pltpu.VMEM_SHARED((NP, EMB), jnp.float32),
            pltpu.SemaphoreType.DMA((2,)),
            pltpu.SemaphoreType.DMA((2,)),
            pltpu.SemaphoreType.DMA((2,)),
            pltpu.SemaphoreType.DMA((2,)),
            pltpu.SemaphoreType.DMA,
        ],
    )
    def k(h_hbm, ea_hbm, src_hbm, dst_hbm, out_hbm,
          src_v, dst_v, hrow_v, ea_v, zb_v, agg_sh,
          ssem, dsem, gsem, esem, zsem):
        c = lax.axis_index("c")
        s = lax.axis_index("s")
        wid = s * NC + c
        ebase = wid * EPW

        def fire_src(i, b):
            pltpu.async_copy(src_hbm.at[wid, i], src_v.at[b], ssem.at[b])

        def fire_dst(i, b):
            pltpu.async_copy(dst_hbm.at[wid, i], dst_v.at[b], dsem.at[b])

        def wait_idx(sem, b):
            pltpu.make_async_copy(src_hbm.at[0, 0], src_v.at[b],
                                  sem.at[b]).wait()

        def fire_data(i, b):
            pltpu.async_copy(h_hbm.at[src_v.at[b]], hrow_v.at[b], gsem.at[b])
            pltpu.async_copy(ea_hbm.at[pl.ds(ebase + i * CHUNK, CHUNK)],
                             ea_v.at[b], esem.at[b])

        def wait_data(b):
            pltpu.make_async_copy(h_hbm.at[src_v.at[b]], hrow_v.at[b],
                                  gsem.at[b]).wait()
            pltpu.make_async_copy(ea_hbm.at[pl.ds(ebase, CHUNK)],
                                  ea_v.at[b], esem.at[b]).wait()

        fire_src(0, 0)
        fire_src(1, 1)
        fire_dst(0, 0)
        fire_dst(1, 1)

        def zrow(r, _):
            for j in range(EMB // 16):
                zb_v[r, pl.ds(j * 16, 16)] = jnp.zeros((16,), jnp.float32)
            return 0
        lax.fori_loop(0, ZROWS, zrow, 0)
        zd = []
        for t in range(RPS // ZROWS):
            zd.append(pltpu.async_copy(
                zb_v, agg_sh.at[pl.ds(s * RPS + t * ZROWS, ZROWS)], zsem))
        for d in zd:
            d.wait()
        plsc.subcore_barrier()

        wait_idx(ssem, 0)
        fire_data(0, 0)
        wait_idx(ssem, 1)
        fire_data(1, 1)

        def pair(g, _):
            for b in range(2):
                i = 2 * g + b
                inext = jnp.minimum(i + 2, NCHUNK - 1)
                wait_data(b)
                fire_src(inext, b)

                def row(r, _):
                    for j in range(EMB // 16):
                        sl = pl.ds(j * 16, 16)
                        hrow_v[b, r, sl] = jnp.maximum(
                            hrow_v[b, r, sl] + ea_v[b, r, sl], 0.0)
                    return 0
                lax.fori_loop(0, CHUNK, row, 0)
                wait_idx(dsem, b)
                pltpu.sync_copy(hrow_v.at[b], agg_sh.at[dst_v.at[b]], add=True)
                fire_dst(inext, b)
                wait_idx(ssem, b)
                fire_data(inext, b)
            return 0
        lax.fori_loop(0, NPAIR, pair, 0)
        for b in range(2):
            wait_data(b)
            wait_idx(dsem, b)
        plsc.subcore_barrier()

        pltpu.sync_copy(agg_sh.at[pl.ds(s * RPS, RPS)],
                        out_hbm.at[c, pl.ds(s * RPS, RPS)])

    return k(h, ea, src2, dst2)[:, :N]


def _tc_prep(h_prev, vn, batch3):
    def body(h_ref, vn_ref, b_ref, h_out, pooled_out):
        i = pl.program_id(0)
        iota = lax.broadcasted_iota(jnp.int32, (G, 1), 0)
        onehot_t = (b_ref[0] == iota).astype(jnp.float32)
        hv = h_ref[...]
        h_out[...] = hv + lax.dot_general(
            onehot_t, vn_ref[...], (((0,), (0,)), ((), ())),
            preferred_element_type=jnp.float32)
        p = jnp.dot(onehot_t, hv, preferred_element_type=jnp.float32)

        @pl.when(i == 0)
        def _():
            pooled_out[...] = p

        @pl.when(i > 0)
        def _():
            pooled_out[...] += p

    return pl.pallas_call(
        body,
        grid=(GRID,),
        in_specs=[
            pl.BlockSpec((R, EMB), lambda i: (i, 0)),
            pl.BlockSpec((G, EMB), lambda i: (0, 0)),
            pl.BlockSpec((1, 1, R), lambda i: (i, 0, 0)),
        ],
        out_specs=[
            pl.BlockSpec((R, EMB), lambda i: (i, 0)),
            pl.BlockSpec((G, EMB), lambda i: (0, 0)),
        ],
        out_shape=[
            jax.ShapeDtypeStruct((N, EMB), jnp.float32),
            jax.ShapeDtypeStruct((G, EMB), jnp.float32),
        ],
    )(h_prev, vn, batch3)


def _tc_fused(h, agg, eps11, c, pooled_l, vn, m, batch3):
    def body(h_ref, a_ref, e_ref, w1_ref, b1_ref, g1_ref, bb1_ref,
             w2_ref, b2_ref, g2_ref, bb2_ref,
             p_ref, v_ref, vw1_ref, vb1_ref, vg1_ref, vbb1_ref,
             vw2_ref, vb2_ref, vg2_ref, vbb2_ref, b3_ref,
             h_out, pooled_out, vn_out):
        i = pl.program_id(0)
        t = (1.0 + e_ref[0, 0]) * h_ref[...] + a_ref[0] + a_ref[1]
        y = jnp.dot(t, w1_ref[...], preferred_element_type=jnp.float32)
        y = jnp.maximum((y + b1_ref[...]) * (g1_ref[...] * _BN_INV)
                        + bb1_ref[...], 0.0)
        z = jnp.dot(y, w2_ref[...], preferred_element_type=jnp.float32)
        z = jnp.maximum((z + b2_ref[...]) * (g2_ref[...] * _BN_INV)
                        + bb2_ref[...], 0.0)
        vt = p_ref[...] + v_ref[...]
        vy = jnp.dot(vt, vw1_ref[...], preferred_element_type=jnp.float32)
        vy = jnp.maximum((vy + vb1_ref[...]) * (vg1_ref[...] * _BN_INV)
                         + vbb1_ref[...], 0.0)
        vz = jnp.dot(vy, vw2_ref[...], preferred_element_type=jnp.float32)
        vn_next = jnp.maximum((vz + vb2_ref[...]) * (vg2_ref[...] * _BN_INV)
                              + vbb2_ref[...], 0.0)
        vn_out[...] = vn_next
        iota = lax.broadcasted_iota(jnp.int32, (G, 1), 0)
        onehot_t = (b3_ref[0] == iota).astype(jnp.float32)
        h_out[...] = z + lax.dot_general(
            onehot_t, vn_next, (((0,), (0,)), ((), ())),
            preferred_element_type=jnp.float32)
        p = jnp.dot(onehot_t, z, preferred_element_type=jnp.float32)

        @pl.when(i == 0)
        def _():
            pooled_out[...] = p

        @pl.when(i > 0)
        def _():
            pooled_out[...] += p

    full = pl.BlockSpec((G, EMB), lambda i: (0, 0))
    w1s = pl.BlockSpec((EMB, 2 * EMB), lambda i: (0, 0))
    v1s = pl.BlockSpec((1, 2 * EMB), lambda i: (0, 0))
    w2s = pl.BlockSpec((2 * EMB, EMB), lambda i: (0, 0))
    v2s = pl.BlockSpec((1, EMB), lambda i: (0, 0))
    return pl.pallas_call(
        body,
        grid=(GRID,),
        in_specs=[
            pl.BlockSpec((R, EMB), lambda i: (i, 0)),
            pl.BlockSpec((NC, R, EMB), lambda i: (0, i, 0)),
            pl.BlockSpec((1, 1), lambda i: (0, 0)),
            w1s, v1s, v1s, v1s, w2s, v2s, v2s, v2s,
            full, full,
            w1s, v1s, v1s, v1s, w2s, v2s, v2s, v2s,
            pl.BlockSpec((1, 1, R), lambda i: (i, 0, 0)),
        ],
        out_specs=[pl.BlockSpec((R, EMB), lambda i: (i, 0)), full, full],
        out_shape=[
            jax.ShapeDtypeStruct((N, EMB), jnp.float32),
            jax.ShapeDtypeStruct((G, EMB), jnp.float32),
            jax.ShapeDtypeStruct((G, EMB), jnp.float32),
        ],
    )(h, agg, eps11,
      c['W1'], c['b1'].reshape(1, -1), c['bn1_g'].reshape(1, -1),
      c['bn1_b'].reshape(1, -1),
      c['W2'], c['b2'].reshape(1, -1), c['bn_g'].reshape(1, -1),
      c['bn_b'].reshape(1, -1),
      pooled_l, vn,
      m['W1'], m['b1'].reshape(1, -1), m['bn1_g'].reshape(1, -1),
      m['bn1_b'].reshape(1, -1),
      m['W2'], m['b2'].reshape(1, -1), m['bn2_g'].reshape(1, -1),
      m['bn2_b'].reshape(1, -1),
      batch3)


def _tc_last(h, agg, eps11, c, batch3, pooled_list, wf_pad, bf_pad):
    def body(h_ref, a_ref, e_ref, w1_ref, b1_ref, g1_ref, bb1_ref,
             w2_ref, b2_ref, g2_ref, bb2_ref, b3_ref,
             p0, p1, p2, p3, p4, wf_ref, bf_ref,
             ne0, ne1, ne2, ne3, ne4, ne5, hg_ref, pred_ref,
             acc, cnt):
        i = pl.program_id(0)
        t = (1.0 + e_ref[0, 0]) * h_ref[...] + a_ref[0] + a_ref[1]
        y = jnp.dot(t, w1_ref[...], preferred_element_type=jnp.float32)
        y = jnp.maximum((y + b1_ref[...]) * (g1_ref[...] * _BN_INV)
                        + bb1_ref[...], 0.0)
        h5 = jnp.dot(y, w2_ref[...], preferred_element_type=jnp.float32)
        h5 = (h5 + b2_ref[...]) * (g2_ref[...] * _BN_INV) + bb2_ref[...]
        iota = lax.broadcasted_iota(jnp.int32, (G, 1), 0)
        onehot_t = (b3_ref[0] == iota).astype(jnp.float32)
        p = jnp.dot(onehot_t, h5, preferred_element_type=jnp.float32)
        csum = jnp.broadcast_to(jnp.sum(onehot_t, axis=1, keepdims=True),
                                (G, EMB))

        @pl.when(i == 0)
        def _():
            acc[...] = p
            cnt[...] = csum

        @pl.when(i > 0)
        def _():
            acc[...] += p
            cnt[...] += csum

        @pl.when(i == GRID - 1)
        def _():
            inv = 1.0 / jnp.maximum(cnt[...], 1.0)
            ne0[...] = p0[...] * inv
            ne1[...] = p1[...] * inv
            ne2[...] = p2[...] * inv
            ne3[...] = p3[...] * inv
            ne4[...] = p4[...] * inv
            hg = acc[...] * inv
            ne5[...] = hg
            hg_ref[...] = hg
            pred_ref[...] = jnp.dot(hg, wf_ref[...],
                                    preferred_element_type=jnp.float32) + bf_ref[...]

    full = pl.BlockSpec((G, EMB), lambda i: (0, 0))
    return pl.pallas_call(
        body,
        grid=(GRID,),
        in_specs=[
            pl.BlockSpec((R, EMB), lambda i: (i, 0)),
            pl.BlockSpec((NC, R, EMB), lambda i: (0, i, 0)),
            pl.BlockSpec((1, 1), lambda i: (0, 0)),
            pl.BlockSpec((EMB, 2 * EMB), lambda i: (0, 0)),
            pl.BlockSpec((1, 2 * EMB), lambda i: (0, 0)),
            pl.BlockSpec((1, 2 * EMB), lambda i: (0, 0)),
            pl.BlockSpec((1, 2 * EMB), lambda i: (0, 0)),
            pl.BlockSpec((2 * EMB, EMB), lambda i: (0, 0)),
            pl.BlockSpec((1, EMB), lambda i: (0, 0)),
            pl.BlockSpec((1, EMB), lambda i: (0, 0)),
            pl.BlockSpec((1, EMB), lambda i: (0, 0)),
            pl.BlockSpec((1, 1, R), lambda i: (i, 0, 0)),
            full, full, full, full, full,
            pl.BlockSpec((EMB, EMB), lambda i: (0, 0)),
            pl.BlockSpec((1, EMB), lambda i: (0, 0)),
        ],
        out_specs=[full] * 8,
        out_shape=[jax.ShapeDtypeStruct((G, EMB), jnp.float32)] * 8,
        scratch_shapes=[
            pltpu.VMEM((G, EMB), jnp.float32),
            pltpu.VMEM((G, EMB), jnp.float32),
        ],
    )(h, agg, eps11,
      c['W1'], c['b1'].reshape(1, -1), c['bn1_g'].reshape(1, -1),
      c['bn1_b'].reshape(1, -1),
      c['W2'], c['b2'].reshape(1, -1), c['bn_g'].reshape(1, -1),
      c['bn_b'].reshape(1, -1),
      batch3, *pooled_list, wf_pad, bf_pad)


def kernel(x, edge_attr, params, edge_index, batch):
    src = edge_index[0].reshape(NW, NCHUNK, CHUNK)
    dst = edge_index[1].reshape(NW, NCHUNK, CHUNK)
    batch3 = batch.reshape(GRID, 1, R)
    vn = jnp.broadcast_to(params['vn_emb'], (G, EMB))
    wf_pad = jnp.zeros((EMB, EMB), jnp.float32).at[:, :NCLS].set(params['Wf'])
    bf_pad = jnp.zeros((1, EMB), jnp.float32).at[0, :NCLS].set(params['bf'])

    h_in, p = _tc_prep(x, vn, batch3)
    pooled = [p]
    for l in range(NL - 1):
        c = params['convs'][l]
        agg = _edge_phase(h_in, edge_attr, src, dst)
        h_in, p, vn = _tc_fused(h_in, agg, c['eps'].reshape(1, 1), c,
                                pooled[-1], vn, params['vnmlps'][l], batch3)
        pooled.append(p)

    c = params['convs'][NL - 1]
    agg = _edge_phase(h_in, edge_attr, src, dst)
    outs = _tc_last(h_in, agg, c['eps'].reshape(1, 1), c, batch3,
                    pooled, wf_pad, bf_pad)
    ne = tuple(outs[:6])
    h_graph = outs[6]
    pred = outs[7][:, :NCLS]
    return (pred, h_graph, ne)
